# baseline (speedup 1.0000x reference)
import functools

import jax
import jax.numpy as jnp
from jax import lax
from jax.experimental import pallas as pl
from jax.experimental.pallas import tpu as pltpu
from jax.experimental.pallas import tpu_sc as plsc

VOCAB = 2000
EMBED_DIM = 19
TFLAT = VOCAB * EMBED_DIM
N_IDX = 16384 * 50
N_W32 = N_IDX // 2
NC = 2
NS = 16
NW = NC * NS
PER_W = N_W32 // NW
CHUNK = PER_W // 2
RPS = 128
UNROLL = 4
STEPS = CHUNK // (16 * UNROLL)
INV_N = 1.0 / float(N_IDX * EMBED_DIM)


def _finalize_body(part_ref, out_ref):
    total = jnp.sum(part_ref[...], axis=(0, 1), keepdims=True)
    out_ref[...] = jax.nn.sigmoid(total * INV_N)


def _sc_gather_sum(idx_flat, tab_flat):
    mesh = plsc.VectorSubcoreMesh(core_axis_name="c", subcore_axis_name="s")

    @functools.partial(
        pl.kernel,
        mesh=mesh,
        compiler_params=pltpu.CompilerParams(
            needs_layout_passes=False,
            skip_device_barrier=True,
            disable_bounds_checks=True,
            disable_semaphore_checks=True,
        ),
        out_type=jax.ShapeDtypeStruct((NC, 16), jnp.float32),
        scratch_types=[
            pltpu.VMEM((CHUNK,), jnp.int32),
            pltpu.VMEM((CHUNK,), jnp.int32),
            pltpu.VMEM((EMBED_DIM * RPS,), jnp.float32),
            pltpu.VMEM((RPS,), jnp.float32),
            pltpu.VMEM((NS * RPS,), jnp.float32),
            pltpu.VMEM((16,), jnp.float32),
            pltpu.VMEM((NS, 16), jnp.float32),
            pltpu.VMEM_SHARED((NS * RPS,), jnp.float32),
            pltpu.VMEM_SHARED((NS, 16), jnp.float32),
            pltpu.SemaphoreType.DMA,
            pltpu.SemaphoreType.DMA,
            pltpu.SemaphoreType.DMA,
        ],
    )
    def k(idx_hbm, tab_hbm, out_hbm, idx_a, idx_b, tab_v, rs_local, rs_v,
          stage_v, part_v, sh_rs, sh_part, sem0, sem1, semt):
        cid = lax.axis_index("c")
        sid = lax.axis_index("s")
        wid = sid * NC + cid
        base = wid * PER_W
        bufs = (idx_a, idx_b)
        sems = (sem0, sem1)
        cps = [None, None]
        cps[0] = pltpu.async_copy(
            idx_hbm.at[pl.ds(base, CHUNK)], idx_a, sems[0]
        )

        c0 = sid * RPS
        tcps = [
            pltpu.async_copy(
                tab_hbm.at[pl.ds(d * VOCAB + c0, RPS)],
                tab_v.at[pl.ds(d * RPS, RPS)],
                semt,
            )
            for d in range(EMBED_DIM)
        ]
        for cp in tcps:
            cp.wait()
        for g in range(RPS // 16):
            acc = tab_v[pl.ds(g * 16, 16)]
            for d in range(1, EMBED_DIM):
                acc = acc + tab_v[pl.ds(d * RPS + g * 16, 16)]
            rs_local[pl.ds(g * 16, 16)] = acc
        pltpu.sync_copy(rs_local, sh_rs.at[pl.ds(sid * RPS, RPS)])
        plsc.subcore_barrier()
        pltpu.sync_copy(sh_rs, rs_v)

        accs = tuple(jnp.zeros((16,), jnp.float32) for _ in range(2 * UNROLL))
        for ch in range(2):
            if ch == 0:
                cps[1] = pltpu.async_copy(
                    idx_hbm.at[pl.ds(base + CHUNK, CHUNK)], idx_b, sems[1]
                )
            cps[ch].wait()
            buf = bufs[ch]

            def body(i, carry):
                out = []
                for u in range(UNROLL):
                    w = buf[pl.ds((i * UNROLL + u) * 16, 16)]
                    lo = w & 0xFFFF
                    hi = lax.shift_right_logical(w, 16)
                    out.append(carry[2 * u] + plsc.load_gather(rs_v, [lo]))
                    out.append(
                        carry[2 * u + 1] + plsc.load_gather(rs_v, [hi])
                    )
                return tuple(out)

            accs = lax.fori_loop(0, STEPS, body, accs)

        tot = accs[0]
        for u in range(1, 2 * UNROLL):
            tot = tot + accs[u]
        stage_v[...] = tot
        pltpu.sync_copy(stage_v, sh_part.at[sid])
        plsc.subcore_barrier()

        @pl.when(sid == 0)
        def _():
            pltpu.sync_copy(sh_part, part_v)
            tv = part_v[0]
            for i in range(1, NS):
                tv = tv + part_v[i]
            stage_v[...] = tv
            pltpu.sync_copy(stage_v, out_hbm.at[cid])

    return k(idx_flat, tab_flat)


def kernel(movies, table):
    idx16 = movies.T.astype(jnp.int16).reshape(N_W32, 2)
    idx_packed = jax.lax.bitcast_convert_type(idx16, jnp.int32)
    partials = _sc_gather_sum(idx_packed, table.T.reshape(TFLAT))
    out = pl.pallas_call(
        _finalize_body,
        out_shape=jax.ShapeDtypeStruct((1, 1), jnp.float32),
    )(partials)
    return out.reshape(())


# speedup vs baseline: 12.6508x; 12.6508x over previous
import functools

import jax
import jax.numpy as jnp
from jax import lax
from jax.experimental import pallas as pl
from jax.experimental.pallas import tpu as pltpu
from jax.experimental.pallas import tpu_sc as plsc

VOCAB = 2000
EMBED_DIM = 19
TFLAT = VOCAB * EMBED_DIM
N_IDX = 16384 * 50
NC = 2
NS = 16
NW = NC * NS
PER_W = N_IDX // NW
CHUNK = PER_W // 2
RPS = 128
UNROLL = 8
STEPS = CHUNK // (16 * UNROLL)
INV_N = 1.0 / float(N_IDX * EMBED_DIM)


def _finalize_body(part_ref, out_ref):
    total = jnp.sum(part_ref[...], axis=(0, 1), keepdims=True)
    out_ref[...] = jax.nn.sigmoid(total * INV_N)


def _sc_gather_sum(idx_flat, tab_flat):
    mesh = plsc.VectorSubcoreMesh(core_axis_name="c", subcore_axis_name="s")

    @functools.partial(
        pl.kernel,
        mesh=mesh,
        compiler_params=pltpu.CompilerParams(
            needs_layout_passes=False,
            skip_device_barrier=True,
            disable_bounds_checks=True,
            disable_semaphore_checks=True,
        ),
        out_type=jax.ShapeDtypeStruct((NC, 16), jnp.float32),
        scratch_types=[
            pltpu.VMEM((CHUNK,), jnp.int32),
            pltpu.VMEM((CHUNK,), jnp.int32),
            pltpu.VMEM((EMBED_DIM * RPS,), jnp.float32),
            pltpu.VMEM((RPS,), jnp.float32),
            pltpu.VMEM((NS * RPS,), jnp.float32),
            pltpu.VMEM((16,), jnp.float32),
            pltpu.VMEM((NS, 16), jnp.float32),
            pltpu.VMEM_SHARED((NS * RPS,), jnp.float32),
            pltpu.VMEM_SHARED((NS, 16), jnp.float32),
            pltpu.SemaphoreType.DMA,
            pltpu.SemaphoreType.DMA,
            pltpu.SemaphoreType.DMA,
        ],
    )
    def k(idx_hbm, tab_hbm, out_hbm, idx_a, idx_b, tab_v, rs_local, rs_v,
          stage_v, part_v, sh_rs, sh_part, sem0, sem1, semt):
        cid = lax.axis_index("c")
        sid = lax.axis_index("s")
        wid = sid * NC + cid
        base = wid * PER_W
        bufs = (idx_a, idx_b)
        sems = (sem0, sem1)
        cps = [None, None]
        cps[0] = pltpu.async_copy(
            idx_hbm.at[pl.ds(base, CHUNK)], idx_a, sems[0]
        )

        c0 = sid * RPS
        tcps = [
            pltpu.async_copy(
                tab_hbm.at[pl.ds(d * VOCAB + c0, RPS)],
                tab_v.at[pl.ds(d * RPS, RPS)],
                semt,
            )
            for d in range(EMBED_DIM)
        ]
        for cp in tcps:
            cp.wait()
        for g in range(RPS // 16):
            acc = tab_v[pl.ds(g * 16, 16)]
            for d in range(1, EMBED_DIM):
                acc = acc + tab_v[pl.ds(d * RPS + g * 16, 16)]
            rs_local[pl.ds(g * 16, 16)] = acc
        pltpu.sync_copy(rs_local, sh_rs.at[pl.ds(sid * RPS, RPS)])
        plsc.subcore_barrier()
        pltpu.sync_copy(sh_rs, rs_v)

        accs = tuple(jnp.zeros((16,), jnp.float32) for _ in range(UNROLL))
        for ch in range(2):
            if ch == 0:
                cps[1] = pltpu.async_copy(
                    idx_hbm.at[pl.ds(base + CHUNK, CHUNK)], idx_b, sems[1]
                )
            cps[ch].wait()
            buf = bufs[ch]

            @plsc.parallel_loop(0, STEPS, step=1, unroll=2, carry=accs)
            def accs(i, carry):
                out = []
                for u in range(UNROLL):
                    iv = buf[pl.ds((i * UNROLL + u) * 16, 16)]
                    out.append(carry[u] + plsc.load_gather(rs_v, [iv]))
                return tuple(out)

        tot = accs[0]
        for u in range(1, UNROLL):
            tot = tot + accs[u]
        stage_v[...] = tot
        pltpu.sync_copy(stage_v, sh_part.at[sid])
        plsc.subcore_barrier()

        @pl.when(sid == 0)
        def _():
            pltpu.sync_copy(sh_part, part_v)
            tv = part_v[0]
            for i in range(1, NS):
                tv = tv + part_v[i]
            stage_v[...] = tv
            pltpu.sync_copy(stage_v, out_hbm.at[cid])

    return k(idx_flat, tab_flat)


def kernel(movies, table):
    partials = _sc_gather_sum(
        movies.T.reshape(N_IDX), table.T.reshape(TFLAT)
    )
    out = pl.pallas_call(
        _finalize_body,
        out_shape=jax.ShapeDtypeStruct((1, 1), jnp.float32),
    )(partials)
    return out.reshape(())


# speedup vs baseline: 12.6960x; 1.0036x over previous
import functools

import jax
import jax.numpy as jnp
from jax import lax
from jax.experimental import pallas as pl
from jax.experimental.pallas import tpu as pltpu
from jax.experimental.pallas import tpu_sc as plsc

VOCAB = 2000
EMBED_DIM = 19
TFLAT = VOCAB * EMBED_DIM
N_IDX = 16384 * 50
NC = 2
NS = 16
NW = NC * NS
PER_W = N_IDX // NW
CHUNK = PER_W // 2
RPS = 128
UNROLL = 8
STEPS = CHUNK // (16 * UNROLL)
INV_N = 1.0 / float(N_IDX * EMBED_DIM)


def _finalize_body(part_ref, out_ref):
    total = jnp.sum(part_ref[...], axis=(0, 1), keepdims=True)
    out_ref[...] = jax.nn.sigmoid(total * INV_N)


def _sc_gather_sum(idx_flat, tab_flat):
    mesh = plsc.VectorSubcoreMesh(core_axis_name="c", subcore_axis_name="s")

    @functools.partial(
        pl.kernel,
        mesh=mesh,
        compiler_params=pltpu.CompilerParams(needs_layout_passes=False),
        out_type=jax.ShapeDtypeStruct((NC, 16), jnp.float32),
        scratch_types=[
            pltpu.VMEM((CHUNK,), jnp.int32),
            pltpu.VMEM((CHUNK,), jnp.int32),
            pltpu.VMEM((EMBED_DIM * RPS,), jnp.float32),
            pltpu.VMEM((RPS,), jnp.float32),
            pltpu.VMEM((NS * RPS,), jnp.float32),
            pltpu.VMEM((16,), jnp.float32),
            pltpu.VMEM((NS, 16), jnp.float32),
            pltpu.VMEM_SHARED((NS * RPS,), jnp.float32),
            pltpu.VMEM_SHARED((NS, 16), jnp.float32),
            pltpu.SemaphoreType.DMA,
            pltpu.SemaphoreType.DMA,
            pltpu.SemaphoreType.DMA,
        ],
    )
    def k(idx_hbm, tab_hbm, out_hbm, idx_a, idx_b, tab_v, rs_local, rs_v,
          stage_v, part_v, sh_rs, sh_part, sem0, sem1, semt):
        cid = lax.axis_index("c")
        sid = lax.axis_index("s")
        wid = sid * NC + cid
        base = wid * PER_W
        bufs = (idx_a, idx_b)
        sems = (sem0, sem1)
        cps = [None, None]
        cps[0] = pltpu.async_copy(
            idx_hbm.at[pl.ds(base, CHUNK)], idx_a, sems[0]
        )

        c0 = sid * RPS
        tcps = [
            pltpu.async_copy(
                tab_hbm.at[pl.ds(d * VOCAB + c0, RPS)],
                tab_v.at[pl.ds(d * RPS, RPS)],
                semt,
            )
            for d in range(EMBED_DIM)
        ]
        for cp in tcps:
            cp.wait()
        for g in range(RPS // 16):
            acc = tab_v[pl.ds(g * 16, 16)]
            for d in range(1, EMBED_DIM):
                acc = acc + tab_v[pl.ds(d * RPS + g * 16, 16)]
            rs_local[pl.ds(g * 16, 16)] = acc
        pltpu.sync_copy(rs_local, sh_rs.at[pl.ds(sid * RPS, RPS)])
        plsc.subcore_barrier()
        pltpu.sync_copy(sh_rs, rs_v)

        accs = tuple(jnp.zeros((16,), jnp.float32) for _ in range(UNROLL))
        for ch in range(2):
            if ch == 0:
                cps[1] = pltpu.async_copy(
                    idx_hbm.at[pl.ds(base + CHUNK, CHUNK)], idx_b, sems[1]
                )
            cps[ch].wait()
            buf = bufs[ch]

            def body(i, carry):
                out = []
                for u in range(UNROLL):
                    iv = buf[pl.ds((i * UNROLL + u) * 16, 16)]
                    out.append(carry[u] + plsc.load_gather(rs_v, [iv]))
                return tuple(out)

            accs = lax.fori_loop(0, STEPS, body, accs)

        tot = accs[0]
        for u in range(1, UNROLL):
            tot = tot + accs[u]
        stage_v[...] = tot
        pltpu.sync_copy(stage_v, sh_part.at[sid])
        plsc.subcore_barrier()

        @pl.when(sid == 0)
        def _():
            pltpu.sync_copy(sh_part, part_v)
            tv = part_v[0]
            for i in range(1, NS):
                tv = tv + part_v[i]
            stage_v[...] = tv
            pltpu.sync_copy(stage_v, out_hbm.at[cid])

    return k(idx_flat, tab_flat)


def kernel(movies, table):
    partials = _sc_gather_sum(
        movies.T.reshape(N_IDX), table.T.reshape(TFLAT)
    )
    out = pl.pallas_call(
        _finalize_body,
        out_shape=jax.ShapeDtypeStruct((1, 1), jnp.float32),
    )(partials)
    return out.reshape(())
